# baseline (speedup 1.0000x reference)
import functools

import jax
import jax.numpy as jnp
from jax import lax
from jax.experimental import pallas as pl
from jax.experimental.pallas import tpu as pltpu
from jax.experimental.pallas import tpu_sc as plsc

B = 16
N = 2048
C = 512
NWIN = 16
W = 16
STRIDE = 128
L = 16
NC = 2
NS = 16
NW = NC * NS
NPAIR = B * NWIN

SC_SHARE = 64
TC_SHARE = NPAIR - SC_SHARE
PPW = SC_SHARE // NW


def _sc_body(x_hbm, out_hbm, rows_v, acc_v, sem):
    wid = lax.axis_index("s") * NC + lax.axis_index("c")
    base = wid * PPW

    def row_start(k):
        p = base + k
        b = p // NWIN
        i = p - b * NWIN
        return b * N + i * STRIDE

    def fire(k):
        pltpu.async_copy(
            x_hbm.at[pl.ds(row_start(k), W)],
            rows_v.at[lax.rem(k, 2)],
            sem,
        )

    fire(0)

    def pair_body(k, carry):
        @pl.when(k + 1 < PPW)
        def _():
            fire(k + 1)

        buf = lax.rem(k, 2)
        pltpu.make_async_copy(
            x_hbm.at[pl.ds(0, W)], rows_v.at[buf], sem
        ).wait()

        def chunk(c, cr):
            off = c * L
            vals = [rows_v[buf, r, pl.ds(off, L)] for r in range(W)]
            while len(vals) > 1:
                nxt = [vals[i] + vals[i + 1] for i in range(0, len(vals) - 1, 2)]
                if len(vals) % 2:
                    nxt.append(vals[-1])
                vals = nxt
            acc_v[k, pl.ds(off, L)] = vals[0] * (1.0 / W)
            return cr

        lax.fori_loop(0, C // L, chunk, 0)
        return carry

    lax.fori_loop(0, PPW, pair_body, 0)
    pltpu.sync_copy(acc_v, out_hbm.at[pl.ds(base, PPW)])


def _tc_body(x_ref, o_ref):
    o_ref[...] = jnp.mean(x_ref[...], axis=2)


def kernel(x):
    x_flat = x.reshape(B * N, C)

    mesh = plsc.VectorSubcoreMesh(core_axis_name="c", subcore_axis_name="s")
    sc_run = functools.partial(
        pl.kernel,
        mesh=mesh,
        out_type=jax.ShapeDtypeStruct((SC_SHARE, C), jnp.float32),
        scratch_types=[
            pltpu.VMEM((2, W, C), jnp.float32),
            pltpu.VMEM((PPW, C), jnp.float32),
            pltpu.SemaphoreType.DMA,
        ],
    )(_sc_body)
    sc_out = sc_run(x_flat)

    SC_B = SC_SHARE // NWIN
    TCB = 8
    x4 = x.reshape(B, NWIN, STRIDE, C)
    tc_out = pl.pallas_call(
        _tc_body,
        grid=(B // TCB,),
        in_specs=[pl.BlockSpec((TCB, NWIN, W, C), lambda j: (j, 0, 0, 0))],
        out_specs=pl.BlockSpec((TCB, NWIN, C), lambda j: (j, 0, 0)),
        out_shape=jax.ShapeDtypeStruct((B, NWIN, C), jnp.float32),
    )(x4)

    out = lax.dynamic_update_slice(
        tc_out, sc_out.reshape(SC_B, NWIN, C), (0, 0, 0)
    )
    return out


# speedup vs baseline: 1.0222x; 1.0222x over previous
import functools

import jax
import jax.numpy as jnp
from jax import lax
from jax.experimental import pallas as pl
from jax.experimental.pallas import tpu as pltpu
from jax.experimental.pallas import tpu_sc as plsc

B = 16
N = 2048
C = 512
NWIN = 16
W = 16
STRIDE = 128
L = 16
NC = 2
NS = 16
NW = NC * NS
NPAIR = B * NWIN

SC_SHARE = 32
TC_SHARE = NPAIR - SC_SHARE
PPW = SC_SHARE // NW


def _sc_body(x_hbm, out_hbm, rows_v, acc_v, sem):
    wid = lax.axis_index("s") * NC + lax.axis_index("c")
    base = wid * PPW

    def row_start(k):
        p = base + k
        b = p // NWIN
        i = p - b * NWIN
        return b * N + i * STRIDE

    def fire(k):
        pltpu.async_copy(
            x_hbm.at[pl.ds(row_start(k), W)],
            rows_v.at[lax.rem(k, 2)],
            sem,
        )

    fire(0)

    def pair_body(k, carry):
        @pl.when(k + 1 < PPW)
        def _():
            fire(k + 1)

        buf = lax.rem(k, 2)
        pltpu.make_async_copy(
            x_hbm.at[pl.ds(0, W)], rows_v.at[buf], sem
        ).wait()

        def chunk(c, cr):
            off = c * L
            vals = [rows_v[buf, r, pl.ds(off, L)] for r in range(W)]
            while len(vals) > 1:
                nxt = [vals[i] + vals[i + 1] for i in range(0, len(vals) - 1, 2)]
                if len(vals) % 2:
                    nxt.append(vals[-1])
                vals = nxt
            acc_v[k, pl.ds(off, L)] = vals[0] * (1.0 / W)
            return cr

        lax.fori_loop(0, C // L, chunk, 0)
        return carry

    lax.fori_loop(0, PPW, pair_body, 0)
    pltpu.sync_copy(acc_v, out_hbm.at[pl.ds(base, PPW)])


def _tc_body(x_ref, o_ref):
    o_ref[...] = jnp.mean(x_ref[...], axis=2)


def kernel(x):
    x_flat = x.reshape(B * N, C)

    mesh = plsc.VectorSubcoreMesh(core_axis_name="c", subcore_axis_name="s")
    sc_run = functools.partial(
        pl.kernel,
        mesh=mesh,
        out_type=jax.ShapeDtypeStruct((SC_SHARE, C), jnp.float32),
        scratch_types=[
            pltpu.VMEM((2, W, C), jnp.float32),
            pltpu.VMEM((PPW, C), jnp.float32),
            pltpu.SemaphoreType.DMA,
        ],
    )(_sc_body)
    sc_out = sc_run(x_flat)

    SC_B = SC_SHARE // NWIN
    TCB = 8
    x4 = x.reshape(B, NWIN, STRIDE, C)
    tc_out = pl.pallas_call(
        _tc_body,
        grid=(B // TCB,),
        in_specs=[pl.BlockSpec((TCB, NWIN, W, C), lambda j: (j, 0, 0, 0))],
        out_specs=pl.BlockSpec((TCB, NWIN, C), lambda j: (j, 0, 0)),
        out_shape=jax.ShapeDtypeStruct((B, NWIN, C), jnp.float32),
    )(x4)

    out = lax.dynamic_update_slice(
        tc_out, sc_out.reshape(SC_B, NWIN, C), (0, 0, 0)
    )
    return out
